# baseline (speedup 1.0000x reference)
import functools

import jax
import jax.numpy as jnp
from jax.experimental import pallas as pl


def _fold_body(W2_ref, Wc_ref, b2_ref, bc_ref, W2c_ref, bc2_ref, *, per_step):
    i = pl.program_id(0)
    Wc = Wc_ref[...]
    for k in range(per_step):
        prod = jnp.dot(W2_ref[k], Wc, preferred_element_type=jnp.float32)
        W2c_ref[k] = prod.astype(jnp.bfloat16)
        e = i * per_step + k
        b2row = b2_ref[pl.ds(e, 1), :].astype(jnp.bfloat16)
        bc2_ref[k] = jnp.dot(b2row, Wc,
                             preferred_element_type=jnp.float32) + bc_ref[...]


def _moe_body(x_ref, W1_ref, b1_ref, W2c_ref, bc2_ref,
              Wg1_ref, bg1_ref, Wg2_ref, bg2_ref,
              out_ref, scores_ref, *, n_experts):
    xblk = x_ref[...]

    g = jnp.dot(xblk, Wg1_ref[...], preferred_element_type=jnp.float32)
    g = jnp.maximum(g + bg1_ref[...], 0.0)
    logits = jnp.dot(g.astype(jnp.bfloat16), Wg2_ref[...],
                     preferred_element_type=jnp.float32) + bg2_ref[...]
    m = jnp.max(logits, axis=1, keepdims=True)
    ex = jnp.exp(logits - m)
    scores_ref[...] = ex / jnp.sum(ex, axis=1, keepdims=True)

    C = out_ref.shape[1] // n_experts
    for e in range(n_experts):
        h = jnp.dot(xblk, W1_ref[e], preferred_element_type=jnp.float32)
        h = jnp.maximum(h + b1_ref[e:e + 1, :], 0.0)
        o = jnp.dot(h.astype(jnp.bfloat16), W2c_ref[e],
                    preferred_element_type=jnp.float32) + bc2_ref[e]
        out_ref[:, e * C:(e + 1) * C] = o


def kernel(x, W1, b1, W2, b2, Wg1, bg1, Wg2, bg2, Wc, bc):
    B, D = x.shape
    E, _, H = W1.shape
    C = Wc.shape[1]
    BT = 256 if B % 256 == 0 else B
    EF = 1

    bf = jnp.bfloat16
    xb = x.astype(bf)
    W1b, W2b = W1.astype(bf), W2.astype(bf)
    Wg1b, Wg2b, Wcb = Wg1.astype(bf), Wg2.astype(bf), Wc.astype(bf)
    b1b = b1.astype(bf)
    bg1_2 = bg1.reshape(1, D)
    bg2_2 = bg2.reshape(1, E)
    bc_2 = bc.reshape(1, C)

    W2c, bc2 = pl.pallas_call(
        functools.partial(_fold_body, per_step=EF),
        grid=(E // EF,),
        in_specs=[
            pl.BlockSpec((EF, H, H), lambda e: (e, 0, 0)),
            pl.BlockSpec((H, C), lambda e: (0, 0)),
            pl.BlockSpec((E, H), lambda e: (0, 0)),
            pl.BlockSpec((1, C), lambda e: (0, 0)),
        ],
        out_specs=[
            pl.BlockSpec((EF, H, C), lambda e: (e, 0, 0)),
            pl.BlockSpec((EF, 1, C), lambda e: (e, 0, 0)),
        ],
        out_shape=[
            jax.ShapeDtypeStruct((E, H, C), bf),
            jax.ShapeDtypeStruct((E, 1, C), jnp.float32),
        ],
    )(W2b, Wcb, b2, bc_2)

    whole = lambda *dims: pl.BlockSpec(dims, lambda t: (0,) * len(dims))
    out, scores = pl.pallas_call(
        functools.partial(_moe_body, n_experts=E),
        grid=(B // BT,),
        in_specs=[
            pl.BlockSpec((BT, D), lambda t: (t, 0)),
            whole(E, D, H),
            whole(E, H),
            whole(E, H, C),
            whole(E, 1, C),
            whole(D, D),
            whole(1, D),
            whole(D, E),
            whole(1, E),
        ],
        out_specs=[
            pl.BlockSpec((BT, E * C), lambda t: (t, 0)),
            pl.BlockSpec((BT, E), lambda t: (t, 0)),
        ],
        out_shape=[
            jax.ShapeDtypeStruct((B, E * C), jnp.float32),
            jax.ShapeDtypeStruct((B, E), jnp.float32),
        ],
    )(xb, W1b, b1b, W2c, bc2, Wg1b, bg1_2, Wg2b, bg2_2)
    return (out.reshape(B, E, C), scores)


# speedup vs baseline: 1.1284x; 1.1284x over previous
import functools

import jax
import jax.numpy as jnp
from jax.experimental import pallas as pl


def _fold_body(W2_ref, Wc_ref, b2_ref, bc_ref, W2c_ref, bc2_ref, *, per_step):
    i = pl.program_id(0)
    Wc = Wc_ref[...]
    for k in range(per_step):
        prod = jnp.dot(W2_ref[k], Wc, preferred_element_type=jnp.float32)
        W2c_ref[k] = prod.astype(jnp.bfloat16)
        e = i * per_step + k
        b2row = b2_ref[pl.ds(e, 1), :].astype(jnp.bfloat16)
        bc2_ref[k] = jnp.dot(b2row, Wc,
                             preferred_element_type=jnp.float32) + bc_ref[...]


def _moe_body(x_ref, W1_ref, b1_ref, W2c_ref, bc2_ref,
              Wg1_ref, bg1_ref, Wg2_ref, bg2_ref,
              out_ref, scores_ref, *, n_experts):
    xblk = x_ref[...]

    g = jnp.dot(xblk, Wg1_ref[...], preferred_element_type=jnp.float32)
    g = jnp.maximum(g + bg1_ref[...], 0.0)
    logits = jnp.dot(g.astype(jnp.bfloat16), Wg2_ref[...],
                     preferred_element_type=jnp.float32) + bg2_ref[...]
    m = jnp.max(logits, axis=1, keepdims=True)
    ex = jnp.exp(logits - m)
    scores_ref[...] = ex / jnp.sum(ex, axis=1, keepdims=True)

    for e in range(n_experts):
        h = jnp.dot(xblk, W1_ref[e], preferred_element_type=jnp.float32)
        h = jnp.maximum(h + b1_ref[e:e + 1, :], 0.0)
        o = jnp.dot(h.astype(jnp.bfloat16), W2c_ref[e],
                    preferred_element_type=jnp.float32) + bc2_ref[e]
        out_ref[:, e, :] = o


def kernel(x, W1, b1, W2, b2, Wg1, bg1, Wg2, bg2, Wc, bc):
    B, D = x.shape
    E, _, H = W1.shape
    C = Wc.shape[1]
    BT = 128 if B % 128 == 0 else B
    EF = 1

    bf = jnp.bfloat16
    xb = x.astype(bf)
    W1b, W2b = W1.astype(bf), W2.astype(bf)
    Wg1b, Wg2b, Wcb = Wg1.astype(bf), Wg2.astype(bf), Wc.astype(bf)
    b1b = b1.astype(bf)
    bg1_2 = bg1.reshape(1, D)
    bg2_2 = bg2.reshape(1, E)
    bc_2 = bc.reshape(1, C)

    W2c, bc2 = pl.pallas_call(
        functools.partial(_fold_body, per_step=EF),
        grid=(E // EF,),
        in_specs=[
            pl.BlockSpec((EF, H, H), lambda e: (e, 0, 0)),
            pl.BlockSpec((H, C), lambda e: (0, 0)),
            pl.BlockSpec((E, H), lambda e: (0, 0)),
            pl.BlockSpec((1, C), lambda e: (0, 0)),
        ],
        out_specs=[
            pl.BlockSpec((EF, H, C), lambda e: (e, 0, 0)),
            pl.BlockSpec((EF, 1, C), lambda e: (e, 0, 0)),
        ],
        out_shape=[
            jax.ShapeDtypeStruct((E, H, C), bf),
            jax.ShapeDtypeStruct((E, 1, C), jnp.float32),
        ],
    )(W2b, Wcb, b2, bc_2)

    whole = lambda *dims: pl.BlockSpec(dims, lambda t: (0,) * len(dims))
    out, scores = pl.pallas_call(
        functools.partial(_moe_body, n_experts=E),
        grid=(B // BT,),
        in_specs=[
            pl.BlockSpec((BT, D), lambda t: (t, 0)),
            whole(E, D, H),
            whole(E, H),
            whole(E, H, C),
            whole(E, 1, C),
            whole(D, D),
            whole(1, D),
            whole(D, E),
            whole(1, E),
        ],
        out_specs=[
            pl.BlockSpec((BT, E, C), lambda t: (t, 0, 0)),
            pl.BlockSpec((BT, E), lambda t: (t, 0)),
        ],
        out_shape=[
            jax.ShapeDtypeStruct((B, E, C), jnp.float32),
            jax.ShapeDtypeStruct((B, E), jnp.float32),
        ],
    )(xb, W1b, b1, W2c, bc2, Wg1b, bg1_2, Wg2b, bg2_2)
    return (out, scores)


# speedup vs baseline: 1.2087x; 1.0712x over previous
import functools

import jax
import jax.numpy as jnp
from jax.experimental import pallas as pl


def _fold_body(W2_ref, Wc_ref, b2_ref, bc_ref, W2c_ref, bc2_ref, *, per_step):
    i = pl.program_id(0)
    Wc = Wc_ref[...]
    for k in range(per_step):
        prod = jnp.dot(W2_ref[k], Wc, preferred_element_type=jnp.float32)
        W2c_ref[k] = prod.astype(jnp.bfloat16)
        e = i * per_step + k
        b2row = b2_ref[pl.ds(e, 1), :].astype(jnp.bfloat16)
        bc2_ref[k] = jnp.dot(b2row, Wc,
                             preferred_element_type=jnp.float32) + bc_ref[...]


def _moe_body(x_ref, W1_ref, b1_ref, W2c_ref, bc2_ref,
              Wg1_ref, bg1_ref, Wg2_ref, bg2_ref,
              out_ref, scores_ref, *, n_experts):
    xblk = x_ref[...]

    g = jnp.dot(xblk, Wg1_ref[...], preferred_element_type=jnp.float32)
    g = jnp.maximum(g + bg1_ref[...], 0.0)
    logits = jnp.dot(g.astype(jnp.bfloat16), Wg2_ref[...],
                     preferred_element_type=jnp.float32) + bg2_ref[...]
    m = jnp.max(logits, axis=1, keepdims=True)
    ex = jnp.exp(logits - m)
    scores_ref[...] = ex / jnp.sum(ex, axis=1, keepdims=True)

    h_next = jnp.dot(xblk, W1_ref[0], preferred_element_type=jnp.float32)
    for e in range(n_experts):
        h = h_next
        if e + 1 < n_experts:
            h_next = jnp.dot(xblk, W1_ref[e + 1],
                             preferred_element_type=jnp.float32)
        h = jnp.maximum(h + b1_ref[e:e + 1, :], 0.0)
        o = jnp.dot(h.astype(jnp.bfloat16), W2c_ref[e],
                    preferred_element_type=jnp.float32) + bc2_ref[e]
        out_ref[:, e, :] = o


def kernel(x, W1, b1, W2, b2, Wg1, bg1, Wg2, bg2, Wc, bc):
    B, D = x.shape
    E, _, H = W1.shape
    C = Wc.shape[1]
    BT = 256 if B % 256 == 0 else B
    EF = 1

    bf = jnp.bfloat16
    xb = x.astype(bf)
    W1b, W2b = W1.astype(bf), W2.astype(bf)
    Wg1b, Wg2b, Wcb = Wg1.astype(bf), Wg2.astype(bf), Wc.astype(bf)
    b1b = b1.astype(bf)
    bg1_2 = bg1.reshape(1, D)
    bg2_2 = bg2.reshape(1, E)
    bc_2 = bc.reshape(1, C)

    W2c, bc2 = pl.pallas_call(
        functools.partial(_fold_body, per_step=EF),
        grid=(E // EF,),
        in_specs=[
            pl.BlockSpec((EF, H, H), lambda e: (e, 0, 0)),
            pl.BlockSpec((H, C), lambda e: (0, 0)),
            pl.BlockSpec((E, H), lambda e: (0, 0)),
            pl.BlockSpec((1, C), lambda e: (0, 0)),
        ],
        out_specs=[
            pl.BlockSpec((EF, H, C), lambda e: (e, 0, 0)),
            pl.BlockSpec((EF, 1, C), lambda e: (e, 0, 0)),
        ],
        out_shape=[
            jax.ShapeDtypeStruct((E, H, C), bf),
            jax.ShapeDtypeStruct((E, 1, C), jnp.float32),
        ],
    )(W2b, Wcb, b2, bc_2)

    whole = lambda *dims: pl.BlockSpec(dims, lambda t: (0,) * len(dims))
    out, scores = pl.pallas_call(
        functools.partial(_moe_body, n_experts=E),
        grid=(B // BT,),
        in_specs=[
            pl.BlockSpec((BT, D), lambda t: (t, 0)),
            whole(E, D, H),
            whole(E, H),
            whole(E, H, C),
            whole(E, 1, C),
            whole(D, D),
            whole(1, D),
            whole(D, E),
            whole(1, E),
        ],
        out_specs=[
            pl.BlockSpec((BT, E, C), lambda t: (t, 0, 0)),
            pl.BlockSpec((BT, E), lambda t: (t, 0)),
        ],
        out_shape=[
            jax.ShapeDtypeStruct((B, E, C), jnp.float32),
            jax.ShapeDtypeStruct((B, E), jnp.float32),
        ],
    )(xb, W1b, b1, W2c, bc2, Wg1b, bg1_2, Wg2b, bg2_2)
    return (out, scores)


# speedup vs baseline: 1.2686x; 1.0496x over previous
import functools

import jax
import jax.numpy as jnp
from jax.experimental import pallas as pl
from jax.experimental.pallas import tpu as pltpu


def _moe_body(x_ref, W1_ref, b1_ref, W2_ref, b2_ref,
              Wg1_ref, bg1_ref, Wg2_ref, bg2_ref, Wc_ref, bc_ref,
              out_ref, scores_ref, W2c_ref, bc2_ref, *, n_experts):
    E = n_experts
    i = pl.program_id(0)

    @pl.when(i < E)
    def _fold():
        Wc = Wc_ref[...]
        prod = jnp.dot(W2_ref[0], Wc, preferred_element_type=jnp.float32)
        W2c_ref[pl.ds(i, 1)] = prod.astype(jnp.bfloat16)[None]
        b2row = b2_ref[pl.ds(i, 1), :].astype(jnp.bfloat16)
        bc2_ref[pl.ds(i, 1)] = (jnp.dot(b2row, Wc,
                                        preferred_element_type=jnp.float32)
                                + bc_ref[...])[None]

    @pl.when(i >= E)
    def _tokens():
        xblk = x_ref[...]

        g = jnp.dot(xblk, Wg1_ref[...], preferred_element_type=jnp.float32)
        g = jnp.maximum(g + bg1_ref[...], 0.0)
        logits = jnp.dot(g.astype(jnp.bfloat16), Wg2_ref[...],
                         preferred_element_type=jnp.float32) + bg2_ref[...]
        m = jnp.max(logits, axis=1, keepdims=True)
        ex = jnp.exp(logits - m)
        scores_ref[...] = ex / jnp.sum(ex, axis=1, keepdims=True)

        for e in range(E):
            h = jnp.dot(xblk, W1_ref[e], preferred_element_type=jnp.float32)
            h = jnp.maximum(h + b1_ref[e:e + 1, :], 0.0)
            o = jnp.dot(h.astype(jnp.bfloat16), W2c_ref[e],
                        preferred_element_type=jnp.float32) + bc2_ref[e]
            out_ref[:, e, :] = o


def kernel(x, W1, b1, W2, b2, Wg1, bg1, Wg2, bg2, Wc, bc):
    B, D = x.shape
    E, _, H = W1.shape
    C = Wc.shape[1]
    BT = 256 if B % 256 == 0 else B
    T = B // BT

    bf = jnp.bfloat16
    xb = x.astype(bf)
    W1b, W2b = W1.astype(bf), W2.astype(bf)
    Wg1b, Wg2b, Wcb = Wg1.astype(bf), Wg2.astype(bf), Wc.astype(bf)
    bg1_2 = bg1.reshape(1, D)
    bg2_2 = bg2.reshape(1, E)
    bc_2 = bc.reshape(1, C)

    tok = lambda i: jnp.maximum(i - E, 0)
    whole = lambda *dims: pl.BlockSpec(dims, lambda i: (0,) * len(dims))
    out, scores = pl.pallas_call(
        functools.partial(_moe_body, n_experts=E),
        grid=(E + T,),
        in_specs=[
            pl.BlockSpec((BT, D), lambda i: (tok(i), 0)),
            whole(E, D, H),
            whole(E, H),
            pl.BlockSpec((1, H, H),
                         lambda i: (jnp.minimum(i, E - 1), 0, 0)),
            whole(E, H),
            whole(D, D),
            whole(1, D),
            whole(D, E),
            whole(1, E),
            whole(H, C),
            whole(1, C),
        ],
        out_specs=[
            pl.BlockSpec((BT, E, C), lambda i: (tok(i), 0, 0)),
            pl.BlockSpec((BT, E), lambda i: (tok(i), 0)),
        ],
        out_shape=[
            jax.ShapeDtypeStruct((B, E, C), jnp.float32),
            jax.ShapeDtypeStruct((B, E), jnp.float32),
        ],
        scratch_shapes=[
            pltpu.VMEM((E, H, C), bf),
            pltpu.VMEM((E, 1, C), jnp.float32),
        ],
        compiler_params=pltpu.CompilerParams(
            vmem_limit_bytes=63 * 1024 * 1024,
        ),
    )(xb, W1b, b1, W2b, b2, Wg1b, bg1_2, Wg2b, bg2_2, Wcb, bc_2)
    return (out, scores)


# speedup vs baseline: 1.2934x; 1.0195x over previous
import functools

import jax
import jax.numpy as jnp
from jax.experimental import pallas as pl
from jax.experimental.pallas import tpu as pltpu


def _moe_body(x_ref, W1_ref, b1_ref, W2_ref, b2_ref,
              Wg1_ref, bg1_ref, Wg2_ref, bg2_ref, Wc_ref, bc_ref,
              out_ref, scores_ref, W2c_ref, bc2_ref, W1v_ref, w1_sem,
              *, n_experts):
    E = n_experts
    i = pl.program_id(0)
    w1_copy = pltpu.make_async_copy(W1_ref, W1v_ref, w1_sem)

    @pl.when(i == 0)
    def _start_w1():
        w1_copy.start()

    @pl.when(i < E)
    def _fold():
        Wc = Wc_ref[...]
        prod = jnp.dot(W2_ref[0], Wc, preferred_element_type=jnp.float32)
        W2c_ref[pl.ds(i, 1)] = prod.astype(jnp.bfloat16)[None]
        b2row = b2_ref[pl.ds(i, 1), :].astype(jnp.bfloat16)
        bc2_ref[pl.ds(i, 1)] = (jnp.dot(b2row, Wc,
                                        preferred_element_type=jnp.float32)
                                + bc_ref[...])[None]

    @pl.when(i == E)
    def _wait_w1():
        w1_copy.wait()

    @pl.when(i >= E)
    def _tokens():
        W1_ref = W1v_ref
        xblk = x_ref[...]

        g = jnp.dot(xblk, Wg1_ref[...], preferred_element_type=jnp.float32)
        g = jnp.maximum(g + bg1_ref[...], 0.0)
        logits = jnp.dot(g.astype(jnp.bfloat16), Wg2_ref[...],
                         preferred_element_type=jnp.float32) + bg2_ref[...]
        m = jnp.max(logits, axis=1, keepdims=True)
        ex = jnp.exp(logits - m)
        scores_ref[...] = ex / jnp.sum(ex, axis=1, keepdims=True)

        for e in range(E):
            h = jnp.dot(xblk, W1_ref[e], preferred_element_type=jnp.float32)
            h = jnp.maximum(h + b1_ref[e:e + 1, :], 0.0)
            o = jnp.dot(h.astype(jnp.bfloat16), W2c_ref[e],
                        preferred_element_type=jnp.float32) + bc2_ref[e]
            out_ref[:, e, :] = o


def kernel(x, W1, b1, W2, b2, Wg1, bg1, Wg2, bg2, Wc, bc):
    B, D = x.shape
    E, _, H = W1.shape
    C = Wc.shape[1]
    BT = 256 if B % 256 == 0 else B
    T = B // BT

    bf = jnp.bfloat16
    xb = x.astype(bf)
    W1b, W2b = W1.astype(bf), W2.astype(bf)
    Wg1b, Wg2b, Wcb = Wg1.astype(bf), Wg2.astype(bf), Wc.astype(bf)
    bg1_2 = bg1.reshape(1, D)
    bg2_2 = bg2.reshape(1, E)
    bc_2 = bc.reshape(1, C)

    tok = lambda i: jnp.maximum(i - E, 0)
    whole = lambda *dims: pl.BlockSpec(dims, lambda i: (0,) * len(dims))
    out, scores = pl.pallas_call(
        functools.partial(_moe_body, n_experts=E),
        grid=(E + T,),
        in_specs=[
            pl.BlockSpec((BT, D), lambda i: (tok(i), 0)),
            pl.BlockSpec(memory_space=pl.ANY),
            whole(E, H),
            pl.BlockSpec((1, H, H),
                         lambda i: (jnp.minimum(i, E - 1), 0, 0)),
            whole(E, H),
            whole(D, D),
            whole(1, D),
            whole(D, E),
            whole(1, E),
            whole(H, C),
            whole(1, C),
        ],
        out_specs=[
            pl.BlockSpec((BT, E, C), lambda i: (tok(i), 0, 0)),
            pl.BlockSpec((BT, E), lambda i: (tok(i), 0)),
        ],
        out_shape=[
            jax.ShapeDtypeStruct((B, E, C), jnp.float32),
            jax.ShapeDtypeStruct((B, E), jnp.float32),
        ],
        scratch_shapes=[
            pltpu.VMEM((E, H, C), bf),
            pltpu.VMEM((E, 1, C), jnp.float32),
            pltpu.VMEM((E, D, H), bf),
            pltpu.SemaphoreType.DMA,
        ],
        compiler_params=pltpu.CompilerParams(
            vmem_limit_bytes=63 * 1024 * 1024,
        ),
    )(xb, W1b, b1, W2b, b2, Wg1b, bg1_2, Wg2b, bg2_2, Wcb, bc_2)
    return (out, scores)


# speedup vs baseline: 1.2935x; 1.0001x over previous
import functools

import jax
import jax.numpy as jnp
from jax.experimental import pallas as pl
from jax.experimental.pallas import tpu as pltpu


def _moe_body(x_ref, W1_ref, b1_ref, W2_ref, b2_ref,
              Wg1_ref, bg1_ref, Wg2_ref, bg2_ref, Wc_ref, bc_ref,
              out_ref, scores_ref, W2c_ref, bc2_ref, W1v_ref, Wg1v_ref,
              w1_sem, wg1_sem, *, n_experts):
    E = n_experts
    i = pl.program_id(0)
    w1_copy = pltpu.make_async_copy(W1_ref, W1v_ref, w1_sem)
    wg1_copy = pltpu.make_async_copy(Wg1_ref, Wg1v_ref, wg1_sem)

    @pl.when(i == 0)
    def _start_prefetch():
        w1_copy.start()
        wg1_copy.start()

    @pl.when(i < E)
    def _fold():
        Wc = Wc_ref[...]
        prod = jnp.dot(W2_ref[0], Wc, preferred_element_type=jnp.float32)
        W2c_ref[pl.ds(i, 1)] = prod.astype(jnp.bfloat16)[None]

        @pl.when(i == 0)
        def _fold_bias():
            r = jnp.dot(b2_ref[...].astype(jnp.bfloat16), Wc,
                        preferred_element_type=jnp.float32) + bc_ref[...]
            bc2_ref[...] = r[:, None, :]

    @pl.when(i == E)
    def _wait_prefetch():
        w1_copy.wait()
        wg1_copy.wait()

    @pl.when(i >= E)
    def _tokens():
        W1_ref = W1v_ref
        Wg1_ref = Wg1v_ref
        xblk = x_ref[...]

        g = jnp.dot(xblk, Wg1_ref[...], preferred_element_type=jnp.float32)
        g = jnp.maximum(g + bg1_ref[...], 0.0)
        logits = jnp.dot(g.astype(jnp.bfloat16), Wg2_ref[...],
                         preferred_element_type=jnp.float32) + bg2_ref[...]
        m = jnp.max(logits, axis=1, keepdims=True)
        ex = jnp.exp(logits - m)
        scores_ref[...] = ex / jnp.sum(ex, axis=1, keepdims=True)

        for e in range(E):
            h = jnp.dot(xblk, W1_ref[e], preferred_element_type=jnp.float32)
            h = jnp.maximum(h + b1_ref[e:e + 1, :], 0.0)
            o = jnp.dot(h.astype(jnp.bfloat16), W2c_ref[e],
                        preferred_element_type=jnp.float32) + bc2_ref[e]
            out_ref[:, e, :] = o


def kernel(x, W1, b1, W2, b2, Wg1, bg1, Wg2, bg2, Wc, bc):
    B, D = x.shape
    E, _, H = W1.shape
    C = Wc.shape[1]
    BT = 256 if B % 256 == 0 else B
    T = B // BT

    bf = jnp.bfloat16
    xb = x.astype(bf)
    W1b, W2b = W1.astype(bf), W2.astype(bf)
    Wg1b, Wg2b, Wcb = Wg1.astype(bf), Wg2.astype(bf), Wc.astype(bf)
    bg1_2 = bg1.reshape(1, D)
    bg2_2 = bg2.reshape(1, E)
    bc_2 = bc.reshape(1, C)

    tok = lambda i: jnp.maximum(i - E, 0)
    whole = lambda *dims: pl.BlockSpec(dims, lambda i: (0,) * len(dims))
    out, scores = pl.pallas_call(
        functools.partial(_moe_body, n_experts=E),
        grid=(E + T,),
        in_specs=[
            pl.BlockSpec((BT, D), lambda i: (tok(i), 0)),
            pl.BlockSpec(memory_space=pl.ANY),
            whole(E, H),
            pl.BlockSpec((1, H, H),
                         lambda i: (jnp.minimum(i, E - 1), 0, 0)),
            whole(E, H),
            pl.BlockSpec(memory_space=pl.ANY),
            whole(1, D),
            whole(D, E),
            whole(1, E),
            whole(H, C),
            whole(1, C),
        ],
        out_specs=[
            pl.BlockSpec((BT, E, C), lambda i: (tok(i), 0, 0)),
            pl.BlockSpec((BT, E), lambda i: (tok(i), 0)),
        ],
        out_shape=[
            jax.ShapeDtypeStruct((B, E, C), jnp.float32),
            jax.ShapeDtypeStruct((B, E), jnp.float32),
        ],
        scratch_shapes=[
            pltpu.VMEM((E, H, C), bf),
            pltpu.VMEM((E, 1, C), jnp.float32),
            pltpu.VMEM((E, D, H), bf),
            pltpu.VMEM((D, D), bf),
            pltpu.SemaphoreType.DMA,
            pltpu.SemaphoreType.DMA,
        ],
        compiler_params=pltpu.CompilerParams(
            vmem_limit_bytes=63 * 1024 * 1024,
        ),
    )(xb, W1b, b1, W2b, b2, Wg1b, bg1_2, Wg2b, bg2_2, Wcb, bc_2)
    return (out, scores)
